# baseline (speedup 1.0000x reference)
import functools

import jax
import jax.numpy as jnp
from jax import lax
from jax.experimental import pallas as pl
from jax.experimental.pallas import tpu as pltpu
from jax.experimental.pallas import tpu_sc as plsc

ROWS = 128
COLS = 32768
LANES = 16
GROUP = LANES * LANES
TCB = 4096
SEG = COLS // 4
SEGS_PER_ROW = COLS // SEG
NUM_CORES = 2
NUM_SUBCORES = 16
NUM_WORKERS = NUM_CORES * NUM_SUBCORES
ROWS_PER_WORKER = ROWS // NUM_WORKERS
NSEG = ROWS_PER_WORKER * SEGS_PER_ROW

_mesh = plsc.VectorSubcoreMesh(core_axis_name="c", subcore_axis_name="s")


def _mask_body(x_ref, m_ref, o_ref):
    o_ref[...] = jnp.where(m_ref[...] != 0, x_ref[...], 0.0)


_premask = pl.pallas_call(
    _mask_body,
    out_shape=jax.ShapeDtypeStruct((ROWS, COLS), jnp.float32),
    grid=(COLS // TCB,),
    in_specs=[
        pl.BlockSpec((ROWS, TCB), lambda j: (0, j)),
        pl.BlockSpec((ROWS, TCB), lambda j: (0, j)),
    ],
    out_specs=pl.BlockSpec((ROWS, TCB), lambda j: (0, j)),
)


@functools.partial(
    pl.kernel,
    mesh=_mesh,
    compiler_params=pltpu.CompilerParams(needs_layout_passes=False),
    out_type=jax.ShapeDtypeStruct((ROWS, COLS), jnp.float32),
    scratch_types=[
        pltpu.VMEM((SEG,), jnp.float32),
        pltpu.VMEM((SEG,), jnp.float32),
        pltpu.SemaphoreType.DMA,
        pltpu.SemaphoreType.DMA,
        pltpu.SemaphoreType.DMA,
        pltpu.SemaphoreType.DMA,
    ],
)
def _cumsum_sc(x_hbm, out_hbm, xb0, xb1, sx0, sx1, so0, so1):
    wid = lax.axis_index("s") * NUM_CORES + lax.axis_index("c")
    base16 = lax.iota(jnp.int32, LANES) * LANES
    xb, sx, so = [xb0, xb1], [sx0, sx1], [so0, so1]
    row0 = wid * ROWS_PER_WORKER

    def seg_slice(s):
        return (row0 + s // SEGS_PER_ROW, pl.ds((s % SEGS_PER_ROW) * SEG, SEG))

    cx, cout = {}, {}
    cx[0] = pltpu.async_copy(x_hbm.at[seg_slice(0)], xb[0], sx[0])
    carry = jnp.float32(0.0)
    for s in range(NSEG):
        p = s & 1
        cx[s].wait()
        xvb = xb[p]
        if s % SEGS_PER_ROW == 0:
            carry = jnp.float32(0.0)

        def group_body(g, carry, xvb=xvb):
            goff = g * GROUP
            idx = [base16 + (goff + j) for j in range(LANES)]
            cols = [plsc.load_gather(xvb, [idx[j]]) for j in range(LANES)]
            partial = cols[0]
            sums = [partial]
            for j in range(1, LANES):
                partial = partial + cols[j]
                sums.append(partial)
            lane_tot = sums[-1]
            incl = plsc.cumsum(lane_tot)
            excl_pc = incl - lane_tot + carry
            for j in range(LANES):
                plsc.store_scatter(xvb, [idx[j]], sums[j] + excl_pc)
            return carry + jnp.sum(lane_tot)

        half = SEG // GROUP // 2
        carry = lax.fori_loop(0, half, group_body, carry)
        if s + 1 < NSEG:
            if s >= 1:
                cout[s - 1].wait()
            cx[s + 1] = pltpu.async_copy(x_hbm.at[seg_slice(s + 1)], xb[1 - p], sx[1 - p])
        carry = lax.fori_loop(half, SEG // GROUP, group_body, carry)
        cout[s] = pltpu.async_copy(xb[p], out_hbm.at[seg_slice(s)], so[p])
    cout[NSEG - 2].wait()
    cout[NSEG - 1].wait()


def kernel(x, mask):
    return _cumsum_sc(_premask(x, mask.astype(jnp.int8)))


# speedup vs baseline: 1.0109x; 1.0109x over previous
import functools

import jax
import jax.numpy as jnp
from jax import lax
from jax.experimental import pallas as pl
from jax.experimental.pallas import tpu as pltpu
from jax.experimental.pallas import tpu_sc as plsc

ROWS = 128
COLS = 32768
LANES = 16
GROUP = LANES * LANES
TCB = 2048
SEG = COLS // 2
SEGS_PER_ROW = COLS // SEG
NUM_CORES = 2
NUM_SUBCORES = 16
NUM_WORKERS = NUM_CORES * NUM_SUBCORES
ROWS_PER_WORKER = ROWS // NUM_WORKERS
NSEG = ROWS_PER_WORKER * SEGS_PER_ROW

_mesh = plsc.VectorSubcoreMesh(core_axis_name="c", subcore_axis_name="s")


def _mask_body(x_ref, m_ref, o_ref):
    o_ref[...] = jnp.where(m_ref[...] != 0, x_ref[...], 0.0)


_premask = pl.pallas_call(
    _mask_body,
    out_shape=jax.ShapeDtypeStruct((ROWS, COLS), jnp.float32),
    grid=(COLS // TCB,),
    in_specs=[
        pl.BlockSpec((ROWS, TCB), lambda j: (0, j)),
        pl.BlockSpec((ROWS, TCB), lambda j: (0, j)),
    ],
    out_specs=pl.BlockSpec((ROWS, TCB), lambda j: (0, j)),
)


@functools.partial(
    pl.kernel,
    mesh=_mesh,
    compiler_params=pltpu.CompilerParams(needs_layout_passes=False),
    out_type=jax.ShapeDtypeStruct((ROWS, COLS), jnp.float32),
    scratch_types=[
        pltpu.VMEM((SEG,), jnp.float32),
        pltpu.VMEM((SEG,), jnp.float32),
        pltpu.SemaphoreType.DMA,
        pltpu.SemaphoreType.DMA,
        pltpu.SemaphoreType.DMA,
        pltpu.SemaphoreType.DMA,
    ],
)
def _cumsum_sc(x_hbm, out_hbm, xb0, xb1, sx0, sx1, so0, so1):
    wid = lax.axis_index("s") * NUM_CORES + lax.axis_index("c")
    base16 = lax.iota(jnp.int32, LANES) * LANES
    xb, sx, so = [xb0, xb1], [sx0, sx1], [so0, so1]
    row0 = wid * ROWS_PER_WORKER

    def seg_slice(s):
        return (row0 + s // SEGS_PER_ROW, pl.ds((s % SEGS_PER_ROW) * SEG, SEG))

    cx, cout = {}, {}
    cx[0] = pltpu.async_copy(x_hbm.at[seg_slice(0)], xb[0], sx[0])
    carry = jnp.float32(0.0)
    for s in range(NSEG):
        p = s & 1
        cx[s].wait()
        xvb = xb[p]
        if s % SEGS_PER_ROW == 0:
            carry = jnp.float32(0.0)

        def group_body(g, carry, xvb=xvb):
            goff = g * GROUP
            idx = [base16 + (goff + j) for j in range(LANES)]
            cols = [plsc.load_gather(xvb, [idx[j]]) for j in range(LANES)]
            partial = cols[0]
            sums = [partial]
            for j in range(1, LANES):
                partial = partial + cols[j]
                sums.append(partial)
            lane_tot = sums[-1]
            incl = plsc.cumsum(lane_tot)
            excl_pc = incl - lane_tot + carry
            for j in range(LANES):
                plsc.store_scatter(xvb, [idx[j]], sums[j] + excl_pc)
            return carry + jnp.sum(lane_tot)

        half = SEG // GROUP // 2
        carry = lax.fori_loop(0, half, group_body, carry)
        if s + 1 < NSEG:
            if s >= 1:
                cout[s - 1].wait()
            cx[s + 1] = pltpu.async_copy(x_hbm.at[seg_slice(s + 1)], xb[1 - p], sx[1 - p])
        carry = lax.fori_loop(half, SEG // GROUP, group_body, carry)
        cout[s] = pltpu.async_copy(xb[p], out_hbm.at[seg_slice(s)], so[p])
    cout[NSEG - 2].wait()
    cout[NSEG - 1].wait()


def kernel(x, mask):
    return _cumsum_sc(_premask(x, mask.astype(jnp.int8)))
